# Initial kernel scaffold; baseline (speedup 1.0000x reference)
import jax
import jax.numpy as jnp
from jax.experimental import pallas as pl


def kernel(boxes, scores, labels):
    raise NotImplementedError("write your pallas kernel here")


# speedup vs baseline: 19.0916x; 19.0916x over previous
import jax
import jax.numpy as jnp
from jax import lax
from jax.experimental import pallas as pl
from jax.experimental.pallas import tpu as pltpu

N = 20000
ROWS = 160
LANES = 128
NP = ROWS * LANES
MAX_OUT = 300
IOU_THRESH = 0.5
NEG = -1e9


def _nms_body(x1_ref, y1_ref, x2_ref, y2_ref, sc_ref, lab_ref, out_ref,
              bx1_s, by1_s, bx2_s, by2_s, ar_s):
    X1 = x1_ref[:]
    Y1 = y1_ref[:]
    X2 = x2_ref[:]
    Y2 = y2_ref[:]
    SC = sc_ref[:]
    LAB = lab_ref[:]

    mc = jnp.max(jnp.maximum(jnp.maximum(X1, X2), jnp.maximum(Y1, Y2)))
    off = LAB * (mc + 1.0)
    BX1 = X1 + off
    BY1 = Y1 + off
    BX2 = X2 + off
    BY2 = Y2 + off
    bx1_s[:] = BX1
    by1_s[:] = BY1
    bx2_s[:] = BX2
    by2_s[:] = BY2
    ar_s[:] = (BX2 - BX1) * (BY2 - BY1)

    idx2d = lax.broadcasted_iota(jnp.int32, (ROWS, LANES), 0) * LANES + \
        lax.broadcasted_iota(jnp.int32, (ROWS, LANES), 1)
    lane_io = lax.broadcasted_iota(jnp.int32, (1, LANES), 1)

    def step(t, rem):
        m = jnp.max(rem)
        valid = m > -1e8
        idx = jnp.min(jnp.where(rem == m, idx2d, jnp.int32(NP)))
        r = idx // LANES
        c = idx % LANES

        def extract(ref):
            row = ref[pl.ds(r, 1), :]
            return jnp.sum(jnp.where(lane_io == c, row, 0.0))

        bx1v = extract(bx1_s)
        by1v = extract(by1_s)
        bx2v = extract(bx2_s)
        by2v = extract(by2_s)
        arv = extract(ar_s)
        ox1v = extract(x1_ref)
        oy1v = extract(y1_ref)
        ox2v = extract(x2_ref)
        oy2v = extract(y2_ref)

        xx1 = jnp.maximum(bx1v, bx1_s[:])
        yy1 = jnp.maximum(by1v, by1_s[:])
        xx2 = jnp.minimum(bx2v, bx2_s[:])
        yy2 = jnp.minimum(by2v, by2_s[:])
        inter = jnp.maximum(xx2 - xx1, 0.0) * jnp.maximum(yy2 - yy1, 0.0)
        union = arv + ar_s[:] - inter
        iou = inter / jnp.maximum(union, 1e-9)
        supp = (iou > IOU_THRESH) & valid
        rem = jnp.where(supp, jnp.float32(NEG), rem)

        vf = jnp.where(valid, 1.0, 0.0).astype(jnp.float32)
        vals = jnp.zeros((1, LANES), jnp.float32)
        vals = jnp.where(lane_io == 0, ox1v * vf, vals)
        vals = jnp.where(lane_io == 1, oy1v * vf, vals)
        vals = jnp.where(lane_io == 2, ox2v * vf, vals)
        vals = jnp.where(lane_io == 3, oy2v * vf, vals)
        vals = jnp.where(lane_io == 4, m * vf, vals)
        out_ref[pl.ds(t, 1), :] = vals
        return rem

    lax.fori_loop(0, MAX_OUT, step, SC)


def kernel(boxes, scores, labels):
    def pad2d(v, fill):
        v = jnp.concatenate(
            [v, jnp.full((NP - N,), fill, jnp.float32)])
        return v.reshape(ROWS, LANES)

    x1 = pad2d(boxes[:, 0], 0.0)
    y1 = pad2d(boxes[:, 1], 0.0)
    x2 = pad2d(boxes[:, 2], 0.0)
    y2 = pad2d(boxes[:, 3], 0.0)
    sc = pad2d(scores, NEG)
    lab = pad2d(labels.astype(jnp.float32), 0.0)

    out = pl.pallas_call(
        _nms_body,
        out_shape=jax.ShapeDtypeStruct((MAX_OUT, LANES), jnp.float32),
        scratch_shapes=[pltpu.VMEM((ROWS, LANES), jnp.float32)
                        for _ in range(5)],
    )(x1, y1, x2, y2, sc, lab)
    return out[:, :5]
